# Initial kernel scaffold; baseline (speedup 1.0000x reference)
import jax
import jax.numpy as jnp
from jax.experimental import pallas as pl


def kernel(src, duration, emb_loc, emb_dur):
    raise NotImplementedError("write your pallas kernel here")


# speedup vs baseline: 2.2013x; 2.2013x over previous
import functools

import jax
import jax.numpy as jnp
from jax import lax
from jax.experimental import pallas as pl
from jax.experimental.pallas import tpu as pltpu
from jax.experimental.pallas import tpu_sc as plsc

_D = 32
_NW = 32
_CHUNK = 128


def _emb_sum_sc(src_flat, dur_flat, emb_loc, emb_dur, *, n):
    per_w = n // _NW
    n_chunks = per_w // _CHUNK
    mesh = plsc.VectorSubcoreMesh(core_axis_name="c", subcore_axis_name="s")

    @functools.partial(
        pl.kernel,
        out_type=jax.ShapeDtypeStruct((n, _D), jnp.float32),
        mesh=mesh,
        scratch_types=[
            pltpu.VMEM((per_w,), jnp.int32),
            pltpu.VMEM((per_w,), jnp.int32),
            pltpu.VMEM((_CHUNK, _D), jnp.float32),
            pltpu.VMEM((_CHUNK, _D), jnp.float32),
            pltpu.SemaphoreType.DMA,
            pltpu.SemaphoreType.DMA,
        ],
        compiler_params=pltpu.CompilerParams(use_tc_tiling_on_sc=False),
    )
    def emb_kernel(src_hbm, dur_hbm, loc_hbm, durtab_hbm, out_hbm,
                   sidx, didx, arows, brows, sem_a, sem_b):
        wid = lax.axis_index("s") * 2 + lax.axis_index("c")
        base = wid * per_w
        pltpu.sync_copy(src_hbm.at[pl.ds(base, per_w)], sidx)
        pltpu.sync_copy(dur_hbm.at[pl.ds(base, per_w)], didx)

        @pl.loop(0, n_chunks)
        def _(ci):
            off = ci * _CHUNK
            ca = pltpu.async_copy(
                loc_hbm.at[sidx.at[pl.ds(off, _CHUNK)]], arows, sem_a)
            cb = pltpu.async_copy(
                durtab_hbm.at[didx.at[pl.ds(off, _CHUNK)]], brows, sem_b)
            ca.wait()
            cb.wait()

            @pl.loop(0, _CHUNK)
            def _(r):
                arows[r, pl.ds(0, 16)] = (
                    arows[r, pl.ds(0, 16)] + brows[r, pl.ds(0, 16)])
                arows[r, pl.ds(16, 16)] = (
                    arows[r, pl.ds(16, 16)] + brows[r, pl.ds(16, 16)])

            pltpu.sync_copy(arows, out_hbm.at[pl.ds(base + off, _CHUNK)])

    return emb_kernel(src_flat, dur_flat, emb_loc, emb_dur)


def kernel(src, duration, emb_loc, emb_dur):
    b, l = src.shape
    n = b * l
    out = _emb_sum_sc(
        src.reshape(n).astype(jnp.int32),
        duration.reshape(n).astype(jnp.int32),
        emb_loc, emb_dur, n=n)
    return out.reshape(b, l, _D)


# speedup vs baseline: 2.2595x; 1.0265x over previous
import functools

import jax
import jax.numpy as jnp
from jax import lax
from jax.experimental import pallas as pl
from jax.experimental.pallas import tpu as pltpu
from jax.experimental.pallas import tpu_sc as plsc

_D = 32
_NW = 32
_CHUNK = 128
_NBUF = 4


def _emb_sum_sc(src_flat, dur_flat, emb_loc, emb_dur, *, n):
    per_w = n // _NW
    n_chunks = per_w // _CHUNK
    mesh = plsc.VectorSubcoreMesh(core_axis_name="c", subcore_axis_name="s")

    @functools.partial(
        pl.kernel,
        out_type=jax.ShapeDtypeStruct((n, _D), jnp.float32),
        mesh=mesh,
        scratch_types=[
            pltpu.VMEM((per_w,), jnp.int32),
            pltpu.VMEM((per_w,), jnp.int32),
            pltpu.VMEM((_NBUF, _CHUNK, _D), jnp.float32),
            pltpu.VMEM((_NBUF, _CHUNK, _D), jnp.float32),
            pltpu.VMEM((_NBUF, _CHUNK, _D), jnp.float32),
            pltpu.SemaphoreType.DMA((_NBUF,)),
            pltpu.SemaphoreType.DMA((_NBUF,)),
        ],
        compiler_params=pltpu.CompilerParams(use_tc_tiling_on_sc=False),
    )
    def emb_kernel(src_hbm, dur_hbm, loc_hbm, durtab_hbm, out_hbm,
                   sidx, didx, arows, brows, orows, sem_g, sem_o):
        wid = lax.axis_index("s") * 2 + lax.axis_index("c")
        base = wid * per_w
        pltpu.sync_copy(src_hbm.at[pl.ds(base, per_w)], sidx)
        pltpu.sync_copy(dur_hbm.at[pl.ds(base, per_w)], didx)

        def issue_gathers(b, c):
            off = c * _CHUNK
            pltpu.async_copy(
                loc_hbm.at[sidx.at[pl.ds(off, _CHUNK)]], arows.at[b],
                sem_g.at[b])
            pltpu.async_copy(
                durtab_hbm.at[didx.at[pl.ds(off, _CHUNK)]], brows.at[b],
                sem_g.at[b])

        def wait_gathers(b):
            pltpu.make_async_copy(
                loc_hbm.at[sidx.at[pl.ds(0, _CHUNK)]], arows.at[b],
                sem_g.at[b]).wait()
            pltpu.make_async_copy(
                durtab_hbm.at[didx.at[pl.ds(0, _CHUNK)]], brows.at[b],
                sem_g.at[b]).wait()

        def wait_out(b):
            pltpu.make_async_copy(
                orows.at[b], out_hbm.at[pl.ds(base, _CHUNK)],
                sem_o.at[b]).wait()

        for b in range(_NBUF):
            issue_gathers(b, b)

        @pl.loop(0, n_chunks, step=_NBUF)
        def _(c0):
            for b in range(_NBUF):
                c = c0 + b
                wait_gathers(b)

                @pl.when(c >= _NBUF)
                def _():
                    wait_out(b)

                @pl.loop(0, _CHUNK)
                def _(r):
                    orows[b, r, pl.ds(0, 16)] = (
                        arows[b, r, pl.ds(0, 16)] + brows[b, r, pl.ds(0, 16)])
                    orows[b, r, pl.ds(16, 16)] = (
                        arows[b, r, pl.ds(16, 16)] + brows[b, r, pl.ds(16, 16)])

                pltpu.async_copy(
                    orows.at[b], out_hbm.at[pl.ds(base + c * _CHUNK, _CHUNK)],
                    sem_o.at[b])

                @pl.when(c + _NBUF < n_chunks)
                def _():
                    issue_gathers(b, c + _NBUF)

        for b in range(_NBUF):
            wait_out(b)

    return emb_kernel(src_flat, dur_flat, emb_loc, emb_dur)


def kernel(src, duration, emb_loc, emb_dur):
    b, l = src.shape
    n = b * l
    out = _emb_sum_sc(
        src.reshape(n).astype(jnp.int32),
        duration.reshape(n).astype(jnp.int32),
        emb_loc, emb_dur, n=n)
    return out.reshape(b, l, _D)


# speedup vs baseline: 2.2617x; 1.0010x over previous
import functools

import jax
import jax.numpy as jnp
from jax import lax
from jax.experimental import pallas as pl
from jax.experimental.pallas import tpu as pltpu
from jax.experimental.pallas import tpu_sc as plsc

_D = 32
_NW = 32
_CHUNK = 128
_NBUF = 4


def _emb_sum_sc(src_flat, dur_flat, emb_loc, emb_dur, *, n):
    per_w = n // _NW
    n_chunks = per_w // _CHUNK
    mesh = plsc.VectorSubcoreMesh(core_axis_name="c", subcore_axis_name="s")

    @functools.partial(
        pl.kernel,
        out_type=jax.ShapeDtypeStruct((n, _D), jnp.float32),
        mesh=mesh,
        scratch_types=[
            pltpu.VMEM((per_w,), jnp.int32),
            pltpu.VMEM((per_w,), jnp.int32),
            pltpu.VMEM((_NBUF, _CHUNK, _D), jnp.float32),
            pltpu.VMEM((_NBUF, _CHUNK, _D), jnp.float32),
            pltpu.VMEM((_NBUF, _CHUNK, _D), jnp.float32),
            pltpu.SemaphoreType.DMA((_NBUF,)),
            pltpu.SemaphoreType.DMA((_NBUF,)),
        ],
        compiler_params=pltpu.CompilerParams(use_tc_tiling_on_sc=False),
    )
    def emb_kernel(src_hbm, dur_hbm, loc_hbm, durtab_hbm, out_hbm,
                   sidx, didx, arows, brows, orows, sem_g, sem_o):
        wid = lax.axis_index("s") * 2 + lax.axis_index("c")
        base = wid * per_w
        pltpu.sync_copy(src_hbm.at[pl.ds(base, per_w)], sidx)
        pltpu.sync_copy(dur_hbm.at[pl.ds(base, per_w)], didx)

        def issue_gathers(b, c):
            off = c * _CHUNK
            pltpu.async_copy(
                loc_hbm.at[sidx.at[pl.ds(off, _CHUNK)]], arows.at[b],
                sem_g.at[b])
            pltpu.async_copy(
                durtab_hbm.at[didx.at[pl.ds(off, _CHUNK)]], brows.at[b],
                sem_g.at[b])

        def wait_gathers(b):
            pltpu.make_async_copy(
                loc_hbm.at[sidx.at[pl.ds(0, _CHUNK)]], arows.at[b],
                sem_g.at[b]).wait()
            pltpu.make_async_copy(
                durtab_hbm.at[didx.at[pl.ds(0, _CHUNK)]], brows.at[b],
                sem_g.at[b]).wait()

        def wait_out(b):
            pltpu.make_async_copy(
                orows.at[b], out_hbm.at[pl.ds(base, _CHUNK)],
                sem_o.at[b]).wait()

        for b in range(_NBUF):
            issue_gathers(b, b)

        @pl.loop(0, n_chunks, step=_NBUF)
        def _(c0):
            for b in range(_NBUF):
                c = c0 + b
                wait_gathers(b)

                @pl.when(c >= _NBUF)
                def _():
                    wait_out(b)

                @pl.loop(0, _CHUNK, step=8)
                def _(r0):
                    for dr in range(8):
                        for j in (0, 16):
                            orows[b, r0 + dr, pl.ds(j, 16)] = (
                                arows[b, r0 + dr, pl.ds(j, 16)]
                                + brows[b, r0 + dr, pl.ds(j, 16)])

                pltpu.async_copy(
                    orows.at[b], out_hbm.at[pl.ds(base + c * _CHUNK, _CHUNK)],
                    sem_o.at[b])

                @pl.when(c + _NBUF < n_chunks)
                def _():
                    issue_gathers(b, c + _NBUF)

        for b in range(_NBUF):
            wait_out(b)

    return emb_kernel(src_flat, dur_flat, emb_loc, emb_dur)


def kernel(src, duration, emb_loc, emb_dur):
    b, l = src.shape
    n = b * l
    out = _emb_sum_sc(
        src.reshape(n).astype(jnp.int32),
        duration.reshape(n).astype(jnp.int32),
        emb_loc, emb_dur, n=n)
    return out.reshape(b, l, _D)


# speedup vs baseline: 2.5160x; 1.1124x over previous
import dataclasses
import functools

import jax
import jax.numpy as jnp
from jax import lax
from jax.experimental import pallas as pl
from jax.experimental.pallas import tpu as pltpu
from jax.experimental.pallas import tpu_sc as plsc

_D = 32
_NW = 32
_CHUNK = 128
_NBUF = 4
_LANES = 16


def _sc_compiler_params():
    cp = pltpu.CompilerParams(use_tc_tiling_on_sc=False)
    if "needs_layout_passes" in pltpu.CompilerParams.__dataclass_fields__:
        cp = dataclasses.replace(cp, needs_layout_passes=False)
    return cp


def _emb_sum_sc(srcT, durT, emb_loc, emb_durT, *, seq_len, batch):
    dur_vocab = emb_durT.shape[1]
    mesh = plsc.VectorSubcoreMesh(core_axis_name="c", subcore_axis_name="s")

    @functools.partial(
        pl.kernel,
        out_type=jax.ShapeDtypeStruct((seq_len, _D, batch), jnp.float32),
        mesh=mesh,
        scratch_types=[
            pltpu.VMEM((seq_len, _CHUNK), jnp.int32),
            pltpu.VMEM((seq_len, _CHUNK), jnp.int32),
            pltpu.VMEM((_D, dur_vocab), jnp.float32),
            pltpu.VMEM((_NBUF, _CHUNK, _D), jnp.float32),
            pltpu.VMEM((_NBUF, _D, _CHUNK), jnp.float32),
            pltpu.SemaphoreType.DMA((_NBUF,)),
            pltpu.SemaphoreType.DMA((_NBUF,)),
        ],
        compiler_params=_sc_compiler_params(),
    )
    def emb_kernel(src_hbm, dur_hbm, loc_hbm, durtab_hbm, out_hbm,
                   sidx, didx, durtab, arows, orows, sem_g, sem_o):
        wid = lax.axis_index("s") * 2 + lax.axis_index("c")
        col0 = wid * _CHUNK
        pltpu.sync_copy(src_hbm.at[:, pl.ds(col0, _CHUNK)], sidx)
        pltpu.sync_copy(dur_hbm.at[:, pl.ds(col0, _CHUNK)], didx)
        pltpu.sync_copy(durtab_hbm, durtab)

        def issue_gather(b, l):
            pltpu.async_copy(
                loc_hbm.at[sidx.at[l]], arows.at[b], sem_g.at[b])

        def wait_gather(b):
            pltpu.make_async_copy(
                loc_hbm.at[sidx.at[0]], arows.at[b], sem_g.at[b]).wait()

        def wait_out(b):
            pltpu.make_async_copy(
                orows.at[b], out_hbm.at[0, :, pl.ds(col0, _CHUNK)],
                sem_o.at[b]).wait()

        iota = lax.iota(jnp.int32, _LANES)

        for b in range(_NBUF):
            issue_gather(b, b)

        @pl.loop(0, seq_len, step=_NBUF)
        def _(l0):
            for b in range(_NBUF):
                l = l0 + b
                wait_gather(b)

                @pl.when(l >= _NBUF)
                def _():
                    wait_out(b)

                @pl.loop(0, _CHUNK, step=_LANES)
                def _(g0):
                    rowv = g0 + iota
                    dvec = didx[l, pl.ds(g0, _LANES)]
                    for j in range(_D):
                        a = plsc.load_gather(
                            arows.at[b], [rowv, jnp.full((_LANES,), j, jnp.int32)])
                        t = plsc.load_gather(
                            durtab, [jnp.full((_LANES,), j, jnp.int32), dvec])
                        orows[b, j, pl.ds(g0, _LANES)] = a + t

                pltpu.async_copy(
                    orows.at[b], out_hbm.at[l, :, pl.ds(col0, _CHUNK)],
                    sem_o.at[b])

                @pl.when(l + _NBUF < seq_len)
                def _():
                    issue_gather(b, l + _NBUF)

        for b in range(_NBUF):
            wait_out(b)

    return emb_kernel(srcT, durT, emb_loc, emb_durT)


def kernel(src, duration, emb_loc, emb_dur):
    b, l = src.shape
    out_t = _emb_sum_sc(
        jnp.transpose(src).astype(jnp.int32),
        jnp.transpose(duration).astype(jnp.int32),
        emb_loc,
        jnp.transpose(emb_dur),
        seq_len=l, batch=b)
    return jnp.transpose(out_t, (2, 0, 1))


# speedup vs baseline: 2.9666x; 1.1791x over previous
import dataclasses
import functools

import jax
import jax.numpy as jnp
from jax import lax
from jax.experimental import pallas as pl
from jax.experimental.pallas import tpu as pltpu
from jax.experimental.pallas import tpu_sc as plsc

_D = 32
_NW = 32
_CHUNK = 128
_NBUF = 4
_LANES = 16


def _sc_compiler_params():
    cp = pltpu.CompilerParams(use_tc_tiling_on_sc=False)
    if "needs_layout_passes" in pltpu.CompilerParams.__dataclass_fields__:
        cp = dataclasses.replace(cp, needs_layout_passes=False)
    return cp


def _emb_sum_sc(srcT, durT, emb_loc, emb_durT, *, seq_len, batch):
    dur_vocab = emb_durT.shape[1]
    mesh = plsc.VectorSubcoreMesh(core_axis_name="c", subcore_axis_name="s")

    @functools.partial(
        pl.kernel,
        out_type=jax.ShapeDtypeStruct((seq_len, _D, batch), jnp.float32),
        mesh=mesh,
        scratch_types=[
            pltpu.VMEM((seq_len, _CHUNK), jnp.int32),
            pltpu.VMEM((seq_len, _CHUNK), jnp.int32),
            pltpu.VMEM((_D, dur_vocab), jnp.float32),
            pltpu.VMEM((_NBUF, _CHUNK, _D), jnp.float32),
            pltpu.VMEM((_D, _CHUNK + 1), jnp.float32),
            pltpu.VMEM((_NBUF, _D, _CHUNK), jnp.float32),
            pltpu.SemaphoreType.DMA((_NBUF,)),
            pltpu.SemaphoreType.DMA((_NBUF,)),
        ],
        compiler_params=_sc_compiler_params(),
    )
    def emb_kernel(src_hbm, dur_hbm, loc_hbm, durtab_hbm, out_hbm,
                   sidx, didx, durtab, arows, apad, orows, sem_g, sem_o):
        wid = lax.axis_index("s") * 2 + lax.axis_index("c")
        col0 = wid * _CHUNK
        pltpu.sync_copy(src_hbm.at[:, pl.ds(col0, _CHUNK)], sidx)
        pltpu.sync_copy(dur_hbm.at[:, pl.ds(col0, _CHUNK)], didx)
        pltpu.sync_copy(durtab_hbm, durtab)

        def issue_gather(b, l):
            pltpu.async_copy(
                loc_hbm.at[sidx.at[l]], arows.at[b], sem_g.at[b])

        def wait_gather(b):
            pltpu.make_async_copy(
                loc_hbm.at[sidx.at[0]], arows.at[b], sem_g.at[b]).wait()

        def wait_out(b):
            pltpu.make_async_copy(
                orows.at[b], out_hbm.at[0, :, pl.ds(col0, _CHUNK)],
                sem_o.at[b]).wait()

        iota = lax.iota(jnp.int32, _LANES)

        for b in range(_NBUF):
            issue_gather(b, b)

        @pl.loop(0, seq_len, step=_NBUF)
        def _(l0):
            for b in range(_NBUF):
                l = l0 + b
                wait_gather(b)

                @pl.when(l >= _NBUF)
                def _():
                    wait_out(b)

                @pl.loop(0, _CHUNK, step=8)
                def _(r0):
                    for dr in range(8):
                        r = r0 + dr
                        rsplat = jnp.full((_LANES,), 0, jnp.int32) + r
                        for jh in (0, 16):
                            v = arows[b, r, pl.ds(jh, _LANES)]
                            plsc.store_scatter(apad, [jh + iota, rsplat], v)

                @pl.loop(0, _CHUNK, step=_LANES)
                def _(g0):
                    rowv = g0 + iota
                    dvec = didx[l, pl.ds(g0, _LANES)]
                    for j in range(_D):
                        jv = jnp.full((_LANES,), j, jnp.int32)
                        t = plsc.load_gather(durtab, [jv, dvec])
                        cur = plsc.load_gather(apad, [jv, rowv])
                        orows[b, j, pl.ds(g0, _LANES)] = cur + t

                pltpu.async_copy(
                    orows.at[b], out_hbm.at[l, :, pl.ds(col0, _CHUNK)],
                    sem_o.at[b])

                @pl.when(l + _NBUF < seq_len)
                def _():
                    issue_gather(b, l + _NBUF)

        for b in range(_NBUF):
            wait_out(b)

    return emb_kernel(srcT, durT, emb_loc, emb_durT)


def kernel(src, duration, emb_loc, emb_dur):
    b, l = src.shape
    out_t = _emb_sum_sc(
        jnp.transpose(src).astype(jnp.int32),
        jnp.transpose(duration).astype(jnp.int32),
        emb_loc,
        jnp.transpose(emb_dur),
        seq_len=l, batch=b)
    return jnp.transpose(out_t, (2, 0, 1))
